# baseline (speedup 1.0000x reference)
import functools

import jax
import jax.numpy as jnp
from jax import lax
from jax.experimental import pallas as pl
from jax.experimental.pallas import tpu as pltpu
from jax.experimental.pallas import tpu_sc as plsc

N_NODES = 10000
N_EDGES = 320000
D = 128

NC = 2
NS = 16
NW = NC * NS

E_PER_TILE = N_EDGES // NW
K = 128
C = 80
E_TILE_PAD = C * K
N_PAD = 10240
ROWS_PER_SUB = N_PAD // NS
PAIRS = C // 2


def _spmm_sc(h, src, dst, w, zeros):
    mesh = plsc.VectorSubcoreMesh(
        core_axis_name="c", subcore_axis_name="s", num_cores=NC, num_subcores=NS
    )

    @functools.partial(
        pl.kernel,
        mesh=mesh,
        out_type=jax.ShapeDtypeStruct((NC, N_PAD, D), jnp.float32),
        scratch_types=[
            pltpu.VMEM((C, K), jnp.int32),
            pltpu.VMEM((K,), jnp.int32),
            pltpu.VMEM((K,), jnp.int32),
            pltpu.VMEM((K,), jnp.float32),
            pltpu.VMEM((K,), jnp.float32),
            pltpu.VMEM((K, D), jnp.float32),
            pltpu.VMEM((K, D), jnp.float32),
            pltpu.VMEM_SHARED((N_PAD, D), jnp.float32),
            pltpu.SemaphoreType.DMA,
            pltpu.SemaphoreType.DMA,
            pltpu.SemaphoreType.DMA,
            pltpu.SemaphoreType.DMA,
        ],
    )
    def spmm_kernel(h_hbm, src_hbm, dst_hbm, w_hbm, z_hbm, out_hbm,
                    dst_v, srcb_a, srcb_b, wb_a, wb_b, rows_a, rows_b,
                    acc, sem_ea, sem_eb, sem_ga, sem_gb):
        c = lax.axis_index("c")
        s = lax.axis_index("s")
        tid = s * NC + c

        pltpu.sync_copy(dst_hbm.at[tid], dst_v)
        rsl = pl.ds(s * ROWS_PER_SUB, ROWS_PER_SUB)
        pltpu.sync_copy(z_hbm.at[rsl], acc.at[rsl])
        plsc.subcore_barrier()

        def edge_copies(i, srcb, wb, sem):
            return (pltpu.make_async_copy(src_hbm.at[tid, i], srcb, sem),
                    pltpu.make_async_copy(w_hbm.at[tid, i], wb, sem))

        def start_edges(i, srcb, wb, sem):
            for cp in edge_copies(i, srcb, wb, sem):
                cp.start()

        def wait_edges(i, srcb, wb, sem):
            for cp in edge_copies(i, srcb, wb, sem):
                cp.wait()

        def gather(srcb, rows, sem):
            return pltpu.make_async_copy(h_hbm.at[srcb], rows, sem)

        def scale_and_scatter(i, rows, wb):
            def group_body(g, carry):
                w16 = wb[pl.ds(g * 16, 16)]
                for eo in range(16):
                    e = g * 16 + eo
                    wv = w16[eo]
                    for d0 in range(D // 16):
                        sl = pl.ds(d0 * 16, 16)
                        rows[e, sl] = rows[e, sl] * wv
                return carry

            lax.fori_loop(0, K // 16, group_body, 0)
            pltpu.sync_copy(rows, acc.at[dst_v.at[i]], add=True)

        start_edges(0, srcb_a, wb_a, sem_ea)
        start_edges(1, srcb_b, wb_b, sem_eb)
        wait_edges(0, srcb_a, wb_a, sem_ea)
        gather(srcb_a, rows_a, sem_ga).start()

        def pair_body(j, carry):
            i0 = 2 * j
            i1 = i0 + 1
            nxt = j + 1 < PAIRS

            gather(srcb_a, rows_a, sem_ga).wait()
            wait_edges(i1, srcb_b, wb_b, sem_eb)
            gather(srcb_b, rows_b, sem_gb).start()
            scale_and_scatter(i0, rows_a, wb_a)

            @pl.when(nxt)
            def _():
                start_edges(i0 + 2, srcb_a, wb_a, sem_ea)

            gather(srcb_b, rows_b, sem_gb).wait()

            @pl.when(nxt)
            def _():
                wait_edges(i0 + 2, srcb_a, wb_a, sem_ea)
                gather(srcb_a, rows_a, sem_ga).start()

            scale_and_scatter(i1, rows_b, wb_b)

            @pl.when(nxt)
            def _():
                start_edges(i1 + 2, srcb_b, wb_b, sem_eb)

            return carry

        lax.fori_loop(0, PAIRS, pair_body, 0)
        plsc.subcore_barrier()
        pltpu.sync_copy(acc.at[rsl], out_hbm.at[c, rsl])

    return spmm_kernel(h, src, dst, w, zeros)


ROW_BLK = 2000


def _linear1(x, W, b):
    def body(x_ref, w_ref, b_ref, o_ref):
        o_ref[...] = lax.dot_general(
            x_ref[...], w_ref[...], (((1,), (1,)), ((), ())),
            preferred_element_type=jnp.float32,
        ) + b_ref[...]

    return pl.pallas_call(
        body,
        grid=(N_NODES // ROW_BLK,),
        in_specs=[
            pl.BlockSpec((ROW_BLK, D), lambda i: (i, 0)),
            pl.BlockSpec((D, D), lambda i: (0, 0)),
            pl.BlockSpec((1, D), lambda i: (0, 0)),
        ],
        out_specs=pl.BlockSpec((ROW_BLK, D), lambda i: (i, 0)),
        out_shape=jax.ShapeDtypeStruct((N_NODES, D), jnp.float32),
    )(x, W, b.reshape(1, D))


def _relu_add_linear(p, W, b):
    def body(p0_ref, p1_ref, w_ref, b_ref, o_ref):
        h = jnp.maximum(p0_ref[...] + p1_ref[...], 0.0)
        o_ref[...] = lax.dot_general(
            h, w_ref[...], (((1,), (1,)), ((), ())),
            preferred_element_type=jnp.float32,
        ) + b_ref[...]

    return pl.pallas_call(
        body,
        grid=(N_NODES // ROW_BLK,),
        in_specs=[
            pl.BlockSpec((ROW_BLK, D), lambda i: (i, 0)),
            pl.BlockSpec((ROW_BLK, D), lambda i: (i, 0)),
            pl.BlockSpec((D, D), lambda i: (0, 0)),
            pl.BlockSpec((1, D), lambda i: (0, 0)),
        ],
        out_specs=pl.BlockSpec((ROW_BLK, D), lambda i: (i, 0)),
        out_shape=jax.ShapeDtypeStruct((N_NODES, D), jnp.float32),
    )(p[0], p[1], W, b.reshape(1, D))


def _add_partials(p):
    def body(p0_ref, p1_ref, o_ref):
        o_ref[...] = p0_ref[...] + p1_ref[...]

    return pl.pallas_call(
        body,
        grid=(N_NODES // ROW_BLK,),
        in_specs=[
            pl.BlockSpec((ROW_BLK, D), lambda i: (i, 0)),
            pl.BlockSpec((ROW_BLK, D), lambda i: (i, 0)),
        ],
        out_specs=pl.BlockSpec((ROW_BLK, D), lambda i: (i, 0)),
        out_shape=jax.ShapeDtypeStruct((N_NODES, D), jnp.float32),
    )(p[0], p[1])


def kernel(x, edge_index, edge_weight, W1, b1, W2, b2):
    pad = ((0, 0), (0, E_TILE_PAD - E_PER_TILE))
    src = jnp.pad(edge_index[0].astype(jnp.int32).reshape(NW, E_PER_TILE),
                  pad).reshape(NW, C, K)
    dst = jnp.pad(edge_index[1].astype(jnp.int32).reshape(NW, E_PER_TILE),
                  pad).reshape(NW, C, K)
    w = jnp.pad(edge_weight.astype(jnp.float32).reshape(NW, E_PER_TILE),
                pad).reshape(NW, C, K)
    zeros = jnp.zeros((N_PAD, D), jnp.float32)

    h = _linear1(x, W1, b1)
    p1 = _spmm_sc(h, src, dst, w, zeros)
    h2 = _relu_add_linear((p1[0, :N_NODES], p1[1, :N_NODES]), W2, b2)
    p2 = _spmm_sc(h2, src, dst, w, zeros)
    return _add_partials((p2[0, :N_NODES], p2[1, :N_NODES]))


# speedup vs baseline: 1.0553x; 1.0553x over previous
import functools

import jax
import jax.numpy as jnp
from jax import lax
from jax.experimental import pallas as pl
from jax.experimental.pallas import tpu as pltpu
from jax.experimental.pallas import tpu_sc as plsc

N_NODES = 10000
N_EDGES = 320000
D = 128

NC = 2
NS = 16
NW = NC * NS

E_PER_TILE = N_EDGES // NW
K = 128
C = 80
E_TILE_PAD = C * K
N_PAD = 10240
ROWS_PER_SUB = N_PAD // NS
PAIRS = C // 2


def _spmm_sc(h, src, dst, w, zeros):
    mesh = plsc.VectorSubcoreMesh(
        core_axis_name="c", subcore_axis_name="s", num_cores=NC, num_subcores=NS
    )

    @functools.partial(
        pl.kernel,
        mesh=mesh,
        out_type=jax.ShapeDtypeStruct((NC, N_PAD, D), jnp.float32),
        scratch_types=[
            pltpu.VMEM((C, K), jnp.int32),
            pltpu.VMEM((K,), jnp.int32),
            pltpu.VMEM((K,), jnp.int32),
            pltpu.VMEM((K,), jnp.float32),
            pltpu.VMEM((K,), jnp.float32),
            pltpu.VMEM((K, D), jnp.float32),
            pltpu.VMEM((K, D), jnp.float32),
            pltpu.VMEM_SHARED((N_PAD, D), jnp.float32),
            pltpu.SemaphoreType.DMA,
            pltpu.SemaphoreType.DMA,
            pltpu.SemaphoreType.DMA,
            pltpu.SemaphoreType.DMA,
            pltpu.SemaphoreType.DMA,
            pltpu.SemaphoreType.DMA,
        ],
    )
    def spmm_kernel(h_hbm, src_hbm, dst_hbm, w_hbm, z_hbm, out_hbm,
                    dst_v, srcb_a, srcb_b, wb_a, wb_b, rows_a, rows_b,
                    acc, sem_ea, sem_eb, sem_ga, sem_gb, sem_sa, sem_sb):
        c = lax.axis_index("c")
        s = lax.axis_index("s")
        tid = s * NC + c

        pltpu.sync_copy(dst_hbm.at[tid], dst_v)
        rsl = pl.ds(s * ROWS_PER_SUB, ROWS_PER_SUB)
        pltpu.sync_copy(z_hbm.at[rsl], acc.at[rsl])
        plsc.subcore_barrier()

        def edge_copies(i, srcb, wb, sem):
            return (pltpu.make_async_copy(src_hbm.at[tid, i], srcb, sem),
                    pltpu.make_async_copy(w_hbm.at[tid, i], wb, sem))

        def start_edges(i, srcb, wb, sem):
            for cp in edge_copies(i, srcb, wb, sem):
                cp.start()

        def wait_edges(i, srcb, wb, sem):
            for cp in edge_copies(i, srcb, wb, sem):
                cp.wait()

        def gather(srcb, rows, sem):
            return pltpu.make_async_copy(h_hbm.at[srcb], rows, sem)

        def scale(rows, wb):
            def group_body(g, carry):
                w16 = wb[pl.ds(g * 16, 16)]
                for eo in range(16):
                    e = g * 16 + eo
                    wv = w16[eo]
                    for d0 in range(D // 16):
                        sl = pl.ds(d0 * 16, 16)
                        rows[e, sl] = rows[e, sl] * wv
                return carry

            lax.fori_loop(0, K // 16, group_body, 0)

        class _Scatter:
            def __init__(self, i, rows, sem):
                self._cp = pltpu.make_async_copy(rows, acc.at[dst_v.at[i]],
                                                 sem)

            def start(self):
                self._cp.start(add=True)

            def wait(self):
                self._cp.wait()

        scatter = _Scatter

        start_edges(0, srcb_a, wb_a, sem_ea)
        start_edges(1, srcb_b, wb_b, sem_eb)
        wait_edges(0, srcb_a, wb_a, sem_ea)
        gather(srcb_a, rows_a, sem_ga).start()

        def pair_body(j, carry):
            i0 = 2 * j
            i1 = i0 + 1
            nxt = j + 1 < PAIRS

            gather(srcb_a, rows_a, sem_ga).wait()
            wait_edges(i1, srcb_b, wb_b, sem_eb)

            @pl.when(j > 0)
            def _():
                scatter(i1 - 2, rows_b, sem_sb).wait()

            gather(srcb_b, rows_b, sem_gb).start()
            scale(rows_a, wb_a)
            scatter(i0, rows_a, sem_sa).start()

            @pl.when(nxt)
            def _():
                start_edges(i0 + 2, srcb_a, wb_a, sem_ea)

            gather(srcb_b, rows_b, sem_gb).wait()

            @pl.when(nxt)
            def _():
                wait_edges(i0 + 2, srcb_a, wb_a, sem_ea)
                scatter(i0, rows_a, sem_sa).wait()
                gather(srcb_a, rows_a, sem_ga).start()

            scale(rows_b, wb_b)
            scatter(i1, rows_b, sem_sb).start()

            @pl.when(nxt)
            def _():
                start_edges(i1 + 2, srcb_b, wb_b, sem_eb)

            return carry

        lax.fori_loop(0, PAIRS, pair_body, 0)
        scatter(C - 2, rows_a, sem_sa).wait()
        scatter(C - 1, rows_b, sem_sb).wait()
        plsc.subcore_barrier()
        pltpu.sync_copy(acc.at[rsl], out_hbm.at[c, rsl])

    return spmm_kernel(h, src, dst, w, zeros)


ROW_BLK = 2000


def _linear1(x, W, b):
    def body(x_ref, w_ref, b_ref, o_ref):
        o_ref[...] = lax.dot_general(
            x_ref[...], w_ref[...], (((1,), (1,)), ((), ())),
            preferred_element_type=jnp.float32,
        ) + b_ref[...]

    return pl.pallas_call(
        body,
        grid=(N_NODES // ROW_BLK,),
        in_specs=[
            pl.BlockSpec((ROW_BLK, D), lambda i: (i, 0)),
            pl.BlockSpec((D, D), lambda i: (0, 0)),
            pl.BlockSpec((1, D), lambda i: (0, 0)),
        ],
        out_specs=pl.BlockSpec((ROW_BLK, D), lambda i: (i, 0)),
        out_shape=jax.ShapeDtypeStruct((N_NODES, D), jnp.float32),
    )(x, W, b.reshape(1, D))


def _relu_add_linear(p, W, b):
    def body(p0_ref, p1_ref, w_ref, b_ref, o_ref):
        h = jnp.maximum(p0_ref[...] + p1_ref[...], 0.0)
        o_ref[...] = lax.dot_general(
            h, w_ref[...], (((1,), (1,)), ((), ())),
            preferred_element_type=jnp.float32,
        ) + b_ref[...]

    return pl.pallas_call(
        body,
        grid=(N_NODES // ROW_BLK,),
        in_specs=[
            pl.BlockSpec((ROW_BLK, D), lambda i: (i, 0)),
            pl.BlockSpec((ROW_BLK, D), lambda i: (i, 0)),
            pl.BlockSpec((D, D), lambda i: (0, 0)),
            pl.BlockSpec((1, D), lambda i: (0, 0)),
        ],
        out_specs=pl.BlockSpec((ROW_BLK, D), lambda i: (i, 0)),
        out_shape=jax.ShapeDtypeStruct((N_NODES, D), jnp.float32),
    )(p[0], p[1], W, b.reshape(1, D))


def _add_partials(p):
    def body(p0_ref, p1_ref, o_ref):
        o_ref[...] = p0_ref[...] + p1_ref[...]

    return pl.pallas_call(
        body,
        grid=(N_NODES // ROW_BLK,),
        in_specs=[
            pl.BlockSpec((ROW_BLK, D), lambda i: (i, 0)),
            pl.BlockSpec((ROW_BLK, D), lambda i: (i, 0)),
        ],
        out_specs=pl.BlockSpec((ROW_BLK, D), lambda i: (i, 0)),
        out_shape=jax.ShapeDtypeStruct((N_NODES, D), jnp.float32),
    )(p[0], p[1])


def kernel(x, edge_index, edge_weight, W1, b1, W2, b2):
    pad = ((0, 0), (0, E_TILE_PAD - E_PER_TILE))
    src = jnp.pad(edge_index[0].astype(jnp.int32).reshape(NW, E_PER_TILE),
                  pad).reshape(NW, C, K)
    dst = jnp.pad(edge_index[1].astype(jnp.int32).reshape(NW, E_PER_TILE),
                  pad).reshape(NW, C, K)
    w = jnp.pad(edge_weight.astype(jnp.float32).reshape(NW, E_PER_TILE),
                pad).reshape(NW, C, K)
    zeros = jnp.zeros((N_PAD, D), jnp.float32)

    h = _linear1(x, W1, b1)
    p1 = _spmm_sc(h, src, dst, w, zeros)
    h2 = _relu_add_linear((p1[0, :N_NODES], p1[1, :N_NODES]), W2, b2)
    p2 = _spmm_sc(h2, src, dst, w, zeros)
    return _add_partials((p2[0, :N_NODES], p2[1, :N_NODES]))


# speedup vs baseline: 1.5038x; 1.4250x over previous
import functools

import jax
import jax.numpy as jnp
from jax import lax
from jax.experimental import pallas as pl
from jax.experimental.pallas import tpu as pltpu
from jax.experimental.pallas import tpu_sc as plsc

N_NODES = 10000
N_EDGES = 320000
D = 128

NC = 2
NS = 16
NW = NC * NS

E_PER_TILE = N_EDGES // NW
K = 128
C = 80
E_TILE_PAD = C * K
N_PAD = 10240
ROWS_PER_SUB = N_PAD // NS
PAIRS = C // 2


def _spmm_sc(h, src, dst, w, zeros):
    mesh = plsc.VectorSubcoreMesh(
        core_axis_name="c", subcore_axis_name="s", num_cores=NC, num_subcores=NS
    )

    @functools.partial(
        pl.kernel,
        mesh=mesh,
        out_type=jax.ShapeDtypeStruct((NC, N_PAD, D), jnp.float32),
        scratch_types=[
            pltpu.VMEM((C, K), jnp.int32),
            pltpu.VMEM((K,), jnp.int32),
            pltpu.VMEM((K,), jnp.int32),
            pltpu.VMEM((K,), jnp.float32),
            pltpu.VMEM((K,), jnp.float32),
            pltpu.VMEM((K, D), jnp.float32),
            pltpu.VMEM((K, D), jnp.float32),
            pltpu.VMEM_SHARED((N_PAD, D), jnp.float32),
            pltpu.SemaphoreType.DMA,
            pltpu.SemaphoreType.DMA,
            pltpu.SemaphoreType.DMA,
            pltpu.SemaphoreType.DMA,
            pltpu.SemaphoreType.DMA,
            pltpu.SemaphoreType.DMA,
        ],
    )
    def spmm_kernel(h_hbm, src_hbm, dst_hbm, w_hbm, z_hbm, out_hbm,
                    dst_v, srcb_a, srcb_b, wb_a, wb_b, rows_a, rows_b,
                    acc, sem_ea, sem_eb, sem_ga, sem_gb, sem_sa, sem_sb):
        c = lax.axis_index("c")
        s = lax.axis_index("s")
        tid = s * NC + c

        pltpu.sync_copy(dst_hbm.at[tid], dst_v)
        rsl = pl.ds(s * ROWS_PER_SUB, ROWS_PER_SUB)
        pltpu.sync_copy(z_hbm.at[rsl], acc.at[rsl])
        plsc.subcore_barrier()

        def edge_copies(i, srcb, wb, sem):
            return (pltpu.make_async_copy(src_hbm.at[tid, i], srcb, sem),
                    pltpu.make_async_copy(w_hbm.at[tid, i], wb, sem))

        def start_edges(i, srcb, wb, sem):
            for cp in edge_copies(i, srcb, wb, sem):
                cp.start()

        def wait_edges(i, srcb, wb, sem):
            for cp in edge_copies(i, srcb, wb, sem):
                cp.wait()

        def gather(srcb, rows, sem):
            return pltpu.make_async_copy(h_hbm.at[c].at[srcb], rows, sem)

        def scale(rows, wb):
            def group_body(g, carry):
                w16 = wb[pl.ds(g * 16, 16)]
                for eo in range(16):
                    e = g * 16 + eo
                    wv = w16[eo]
                    for d0 in range(D // 16):
                        sl = pl.ds(d0 * 16, 16)
                        rows[e, sl] = rows[e, sl] * wv
                return carry

            lax.fori_loop(0, K // 16, group_body, 0)

        class _Scatter:
            def __init__(self, i, rows, sem):
                self._cp = pltpu.make_async_copy(rows, acc.at[dst_v.at[i]],
                                                 sem)

            def start(self):
                self._cp.start(add=True)

            def wait(self):
                self._cp.wait()

        scatter = _Scatter

        start_edges(0, srcb_a, wb_a, sem_ea)
        start_edges(1, srcb_b, wb_b, sem_eb)
        wait_edges(0, srcb_a, wb_a, sem_ea)
        gather(srcb_a, rows_a, sem_ga).start()

        def pair_body(j, carry):
            i0 = 2 * j
            i1 = i0 + 1
            nxt = j + 1 < PAIRS

            gather(srcb_a, rows_a, sem_ga).wait()
            wait_edges(i1, srcb_b, wb_b, sem_eb)

            @pl.when(j > 0)
            def _():
                scatter(i1 - 2, rows_b, sem_sb).wait()

            gather(srcb_b, rows_b, sem_gb).start()
            scale(rows_a, wb_a)
            scatter(i0, rows_a, sem_sa).start()

            @pl.when(nxt)
            def _():
                start_edges(i0 + 2, srcb_a, wb_a, sem_ea)

            gather(srcb_b, rows_b, sem_gb).wait()

            @pl.when(nxt)
            def _():
                wait_edges(i0 + 2, srcb_a, wb_a, sem_ea)
                scatter(i0, rows_a, sem_sa).wait()
                gather(srcb_a, rows_a, sem_ga).start()

            scale(rows_b, wb_b)
            scatter(i1, rows_b, sem_sb).start()

            @pl.when(nxt)
            def _():
                start_edges(i1 + 2, srcb_b, wb_b, sem_eb)

            return carry

        lax.fori_loop(0, PAIRS, pair_body, 0)
        scatter(C - 2, rows_a, sem_sa).wait()
        scatter(C - 1, rows_b, sem_sb).wait()
        plsc.subcore_barrier()
        pltpu.sync_copy(acc.at[rsl], out_hbm.at[c, rsl])

    return spmm_kernel(h, src, dst, w, zeros)


ROW_BLK = 2000


def _linear1(x, W, b):
    def body(x_ref, w_ref, b_ref, o_ref):
        o_ref[...] = lax.dot_general(
            x_ref[...], w_ref[...], (((1,), (1,)), ((), ())),
            preferred_element_type=jnp.float32,
        ) + b_ref[...]

    return pl.pallas_call(
        body,
        grid=(N_NODES // ROW_BLK,),
        in_specs=[
            pl.BlockSpec((ROW_BLK, D), lambda i: (i, 0)),
            pl.BlockSpec((D, D), lambda i: (0, 0)),
            pl.BlockSpec((1, D), lambda i: (0, 0)),
        ],
        out_specs=pl.BlockSpec((ROW_BLK, D), lambda i: (i, 0)),
        out_shape=jax.ShapeDtypeStruct((N_NODES, D), jnp.float32),
    )(x, W, b.reshape(1, D))


def _relu_add_linear(p, W, b):
    def body(p0_ref, p1_ref, w_ref, b_ref, o_ref):
        h = jnp.maximum(p0_ref[...] + p1_ref[...], 0.0)
        o_ref[...] = lax.dot_general(
            h, w_ref[...], (((1,), (1,)), ((), ())),
            preferred_element_type=jnp.float32,
        ) + b_ref[...]

    return pl.pallas_call(
        body,
        grid=(N_NODES // ROW_BLK,),
        in_specs=[
            pl.BlockSpec((ROW_BLK, D), lambda i: (i, 0)),
            pl.BlockSpec((ROW_BLK, D), lambda i: (i, 0)),
            pl.BlockSpec((D, D), lambda i: (0, 0)),
            pl.BlockSpec((1, D), lambda i: (0, 0)),
        ],
        out_specs=pl.BlockSpec((ROW_BLK, D), lambda i: (i, 0)),
        out_shape=jax.ShapeDtypeStruct((N_NODES, D), jnp.float32),
    )(p[0], p[1], W, b.reshape(1, D))


def _add_partials(p):
    def body(p0_ref, p1_ref, o_ref):
        o_ref[...] = p0_ref[...] + p1_ref[...]

    return pl.pallas_call(
        body,
        grid=(N_NODES // ROW_BLK,),
        in_specs=[
            pl.BlockSpec((ROW_BLK, D), lambda i: (i, 0)),
            pl.BlockSpec((ROW_BLK, D), lambda i: (i, 0)),
        ],
        out_specs=pl.BlockSpec((ROW_BLK, D), lambda i: (i, 0)),
        out_shape=jax.ShapeDtypeStruct((N_NODES, D), jnp.float32),
    )(p[0], p[1])


def kernel(x, edge_index, edge_weight, W1, b1, W2, b2):
    pad = ((0, 0), (0, E_TILE_PAD - E_PER_TILE))
    src = jnp.pad(edge_index[0].astype(jnp.int32).reshape(NW, E_PER_TILE),
                  pad).reshape(NW, C, K)
    dst = jnp.pad(edge_index[1].astype(jnp.int32).reshape(NW, E_PER_TILE),
                  pad).reshape(NW, C, K)
    w = jnp.pad(edge_weight.astype(jnp.float32).reshape(NW, E_PER_TILE),
                pad).reshape(NW, C, K)
    zeros = jnp.zeros((N_PAD, D), jnp.float32)

    h = _linear1(x, W1, b1)
    p1 = _spmm_sc(jnp.stack([h, h]), src, dst, w, zeros)
    h2 = _relu_add_linear((p1[0, :N_NODES], p1[1, :N_NODES]), W2, b2)
    p2 = _spmm_sc(jnp.stack([h2, h2]), src, dst, w, zeros)
    return _add_partials((p2[0, :N_NODES], p2[1, :N_NODES]))


# speedup vs baseline: 2.0787x; 1.3823x over previous
import functools

import jax
import jax.numpy as jnp
from jax import lax
from jax.experimental import pallas as pl
from jax.experimental.pallas import tpu as pltpu
from jax.experimental.pallas import tpu_sc as plsc

N_NODES = 10000
N_EDGES = 320000
D = 128

NC = 2
NS = 16
NW = NC * NS

E_PER_TILE = N_EDGES // NW
K = 128
C = 80
E_TILE_PAD = C * K
N_PAD = 10240
ROWS_PER_SUB = N_PAD // NS
PAIRS = C // 2


def _spmm_sc(h, src, dst, w, zeros):
    mesh = plsc.VectorSubcoreMesh(
        core_axis_name="c", subcore_axis_name="s", num_cores=NC, num_subcores=NS
    )

    @functools.partial(
        pl.kernel,
        mesh=mesh,
        out_type=jax.ShapeDtypeStruct((NC, N_PAD, D), jnp.float32),
        scratch_types=[
            pltpu.VMEM((C, K), jnp.int32),
            pltpu.VMEM((K,), jnp.int32),
            pltpu.VMEM((K,), jnp.int32),
            pltpu.VMEM((K,), jnp.float32),
            pltpu.VMEM((K,), jnp.float32),
            pltpu.VMEM((K, D), jnp.float32),
            pltpu.VMEM((K, D), jnp.float32),
            pltpu.VMEM_SHARED((N_PAD, D), jnp.float32),
            pltpu.SemaphoreType.DMA,
            pltpu.SemaphoreType.DMA,
            pltpu.SemaphoreType.DMA,
            pltpu.SemaphoreType.DMA,
            pltpu.SemaphoreType.DMA,
            pltpu.SemaphoreType.DMA,
        ],
    )
    def spmm_kernel(h_hbm, src_hbm, dst_hbm, w_hbm, z_hbm, out_hbm,
                    dst_v, srcb_a, srcb_b, wb_a, wb_b, rows_a, rows_b,
                    acc, sem_ea, sem_eb, sem_ga, sem_gb, sem_sa, sem_sb):
        c = lax.axis_index("c")
        s = lax.axis_index("s")
        tid = s * NC + c

        pltpu.sync_copy(dst_hbm.at[tid], dst_v)
        rsl = pl.ds(s * ROWS_PER_SUB, ROWS_PER_SUB)
        pltpu.sync_copy(z_hbm.at[rsl], acc.at[rsl])
        plsc.subcore_barrier()

        def edge_copies(i, srcb, wb, sem):
            return (pltpu.make_async_copy(src_hbm.at[tid, i], srcb, sem),
                    pltpu.make_async_copy(w_hbm.at[tid, i], wb, sem))

        def start_edges(i, srcb, wb, sem):
            for cp in edge_copies(i, srcb, wb, sem):
                cp.start()

        def wait_edges(i, srcb, wb, sem):
            for cp in edge_copies(i, srcb, wb, sem):
                cp.wait()

        hcopy = c * 4 + lax.rem(s, 4)

        def gather(srcb, rows, sem):
            return pltpu.make_async_copy(h_hbm.at[hcopy].at[srcb], rows, sem)

        def scale(rows, wb):
            def group_body(g, carry):
                w16 = wb[pl.ds(g * 16, 16)]
                for eo in range(16):
                    e = g * 16 + eo
                    wv = w16[eo]
                    for d0 in range(D // 16):
                        sl = pl.ds(d0 * 16, 16)
                        rows[e, sl] = rows[e, sl] * wv
                return carry

            lax.fori_loop(0, K // 16, group_body, 0)

        class _Scatter:
            def __init__(self, i, rows, sem):
                self._cp = pltpu.make_async_copy(rows, acc.at[dst_v.at[i]],
                                                 sem)

            def start(self):
                self._cp.start(add=True)

            def wait(self):
                self._cp.wait()

        scatter = _Scatter

        start_edges(0, srcb_a, wb_a, sem_ea)
        start_edges(1, srcb_b, wb_b, sem_eb)
        wait_edges(0, srcb_a, wb_a, sem_ea)
        gather(srcb_a, rows_a, sem_ga).start()

        def pair_body(j, carry):
            i0 = 2 * j
            i1 = i0 + 1
            nxt = j + 1 < PAIRS

            gather(srcb_a, rows_a, sem_ga).wait()
            wait_edges(i1, srcb_b, wb_b, sem_eb)

            @pl.when(j > 0)
            def _():
                scatter(i1 - 2, rows_b, sem_sb).wait()

            gather(srcb_b, rows_b, sem_gb).start()
            scale(rows_a, wb_a)
            scatter(i0, rows_a, sem_sa).start()

            @pl.when(nxt)
            def _():
                start_edges(i0 + 2, srcb_a, wb_a, sem_ea)

            gather(srcb_b, rows_b, sem_gb).wait()

            @pl.when(nxt)
            def _():
                wait_edges(i0 + 2, srcb_a, wb_a, sem_ea)
                scatter(i0, rows_a, sem_sa).wait()
                gather(srcb_a, rows_a, sem_ga).start()

            scale(rows_b, wb_b)
            scatter(i1, rows_b, sem_sb).start()

            @pl.when(nxt)
            def _():
                start_edges(i1 + 2, srcb_b, wb_b, sem_eb)

            return carry

        lax.fori_loop(0, PAIRS, pair_body, 0)
        scatter(C - 2, rows_a, sem_sa).wait()
        scatter(C - 1, rows_b, sem_sb).wait()
        plsc.subcore_barrier()
        pltpu.sync_copy(acc.at[rsl], out_hbm.at[c, rsl])

    return spmm_kernel(h, src, dst, w, zeros)


ROW_BLK = 2000


def _linear1(x, W, b):
    def body(x_ref, w_ref, b_ref, o_ref):
        o_ref[...] = lax.dot_general(
            x_ref[...], w_ref[...], (((1,), (1,)), ((), ())),
            preferred_element_type=jnp.float32,
        ) + b_ref[...]

    return pl.pallas_call(
        body,
        grid=(N_NODES // ROW_BLK,),
        in_specs=[
            pl.BlockSpec((ROW_BLK, D), lambda i: (i, 0)),
            pl.BlockSpec((D, D), lambda i: (0, 0)),
            pl.BlockSpec((1, D), lambda i: (0, 0)),
        ],
        out_specs=pl.BlockSpec((ROW_BLK, D), lambda i: (i, 0)),
        out_shape=jax.ShapeDtypeStruct((N_NODES, D), jnp.float32),
    )(x, W, b.reshape(1, D))


def _relu_add_linear(p, W, b):
    def body(p0_ref, p1_ref, w_ref, b_ref, o_ref):
        h = jnp.maximum(p0_ref[...] + p1_ref[...], 0.0)
        o_ref[...] = lax.dot_general(
            h, w_ref[...], (((1,), (1,)), ((), ())),
            preferred_element_type=jnp.float32,
        ) + b_ref[...]

    return pl.pallas_call(
        body,
        grid=(N_NODES // ROW_BLK,),
        in_specs=[
            pl.BlockSpec((ROW_BLK, D), lambda i: (i, 0)),
            pl.BlockSpec((ROW_BLK, D), lambda i: (i, 0)),
            pl.BlockSpec((D, D), lambda i: (0, 0)),
            pl.BlockSpec((1, D), lambda i: (0, 0)),
        ],
        out_specs=pl.BlockSpec((ROW_BLK, D), lambda i: (i, 0)),
        out_shape=jax.ShapeDtypeStruct((N_NODES, D), jnp.float32),
    )(p[0], p[1], W, b.reshape(1, D))


def _add_partials(p):
    def body(p0_ref, p1_ref, o_ref):
        o_ref[...] = p0_ref[...] + p1_ref[...]

    return pl.pallas_call(
        body,
        grid=(N_NODES // ROW_BLK,),
        in_specs=[
            pl.BlockSpec((ROW_BLK, D), lambda i: (i, 0)),
            pl.BlockSpec((ROW_BLK, D), lambda i: (i, 0)),
        ],
        out_specs=pl.BlockSpec((ROW_BLK, D), lambda i: (i, 0)),
        out_shape=jax.ShapeDtypeStruct((N_NODES, D), jnp.float32),
    )(p[0], p[1])


def kernel(x, edge_index, edge_weight, W1, b1, W2, b2):
    pad = ((0, 0), (0, E_TILE_PAD - E_PER_TILE))
    src = jnp.pad(edge_index[0].astype(jnp.int32).reshape(NW, E_PER_TILE),
                  pad).reshape(NW, C, K)
    dst = jnp.pad(edge_index[1].astype(jnp.int32).reshape(NW, E_PER_TILE),
                  pad).reshape(NW, C, K)
    w = jnp.pad(edge_weight.astype(jnp.float32).reshape(NW, E_PER_TILE),
                pad).reshape(NW, C, K)
    zeros = jnp.zeros((N_PAD, D), jnp.float32)

    h = _linear1(x, W1, b1)
    p1 = _spmm_sc(jnp.stack([h] * 8), src, dst, w, zeros)
    h2 = _relu_add_linear((p1[0, :N_NODES], p1[1, :N_NODES]), W2, b2)
    p2 = _spmm_sc(jnp.stack([h2] * 8), src, dst, w, zeros)
    return _add_partials((p2[0, :N_NODES], p2[1, :N_NODES]))


# speedup vs baseline: 2.1145x; 1.0172x over previous
import functools

import jax
import jax.numpy as jnp
from jax import lax
from jax.experimental import pallas as pl
from jax.experimental.pallas import tpu as pltpu
from jax.experimental.pallas import tpu_sc as plsc

N_NODES = 10000
N_EDGES = 320000
D = 128

NC = 2
NS = 16
NW = NC * NS

E_PER_TILE = N_EDGES // NW
K = 128
C = 80
E_TILE_PAD = C * K
N_PAD = 10240
ROWS_PER_SUB = N_PAD // NS
PAIRS = C // 2


def _spmm_sc(h, src, dst, w, zeros):
    mesh = plsc.VectorSubcoreMesh(
        core_axis_name="c", subcore_axis_name="s", num_cores=NC, num_subcores=NS
    )

    @functools.partial(
        pl.kernel,
        mesh=mesh,
        out_type=jax.ShapeDtypeStruct((NC, N_PAD, D), jnp.float32),
        scratch_types=[
            pltpu.VMEM((C, K), jnp.int32),
            pltpu.VMEM((K,), jnp.int32),
            pltpu.VMEM((K,), jnp.int32),
            pltpu.VMEM((K,), jnp.float32),
            pltpu.VMEM((K,), jnp.float32),
            pltpu.VMEM((K, D), jnp.float32),
            pltpu.VMEM((K, D), jnp.float32),
            pltpu.VMEM_SHARED((N_PAD, D), jnp.float32),
            pltpu.SemaphoreType.DMA,
            pltpu.SemaphoreType.DMA,
            pltpu.SemaphoreType.DMA,
            pltpu.SemaphoreType.DMA,
            pltpu.SemaphoreType.DMA,
            pltpu.SemaphoreType.DMA,
        ],
    )
    def spmm_kernel(h_hbm, src_hbm, dst_hbm, w_hbm, z_hbm, out_hbm,
                    dst_v, srcb_a, srcb_b, wb_a, wb_b, rows_a, rows_b,
                    acc, sem_ea, sem_eb, sem_ga, sem_gb, sem_sa, sem_sb):
        c = lax.axis_index("c")
        s = lax.axis_index("s")
        tid = s * NC + c

        pltpu.sync_copy(dst_hbm.at[tid], dst_v)
        rsl = pl.ds(s * ROWS_PER_SUB, ROWS_PER_SUB)
        pltpu.sync_copy(z_hbm.at[rsl], acc.at[rsl])
        plsc.subcore_barrier()

        def edge_copies(i, srcb, wb, sem):
            return (pltpu.make_async_copy(src_hbm.at[tid, i], srcb, sem),
                    pltpu.make_async_copy(w_hbm.at[tid, i], wb, sem))

        def start_edges(i, srcb, wb, sem):
            for cp in edge_copies(i, srcb, wb, sem):
                cp.start()

        def wait_edges(i, srcb, wb, sem):
            for cp in edge_copies(i, srcb, wb, sem):
                cp.wait()

        hcopy = c * 8 + lax.rem(s, 8)

        def gather(srcb, rows, sem):
            return pltpu.make_async_copy(h_hbm.at[hcopy].at[srcb], rows, sem)

        def scale(rows, wb):
            def group_body(g, carry):
                w16 = wb[pl.ds(g * 16, 16)]
                for eo in range(16):
                    e = g * 16 + eo
                    wv = w16[eo]
                    for d0 in range(D // 16):
                        sl = pl.ds(d0 * 16, 16)
                        rows[e, sl] = rows[e, sl] * wv
                return carry

            lax.fori_loop(0, K // 16, group_body, 0)

        class _Scatter:
            def __init__(self, i, rows, sem):
                self._cp = pltpu.make_async_copy(rows, acc.at[dst_v.at[i]],
                                                 sem)

            def start(self):
                self._cp.start(add=True)

            def wait(self):
                self._cp.wait()

        scatter = _Scatter

        start_edges(0, srcb_a, wb_a, sem_ea)
        start_edges(1, srcb_b, wb_b, sem_eb)
        wait_edges(0, srcb_a, wb_a, sem_ea)
        gather(srcb_a, rows_a, sem_ga).start()

        def pair_body(j, carry):
            i0 = 2 * j
            i1 = i0 + 1
            nxt = j + 1 < PAIRS

            gather(srcb_a, rows_a, sem_ga).wait()
            wait_edges(i1, srcb_b, wb_b, sem_eb)

            @pl.when(j > 0)
            def _():
                scatter(i1 - 2, rows_b, sem_sb).wait()

            gather(srcb_b, rows_b, sem_gb).start()
            scale(rows_a, wb_a)
            scatter(i0, rows_a, sem_sa).start()

            @pl.when(nxt)
            def _():
                start_edges(i0 + 2, srcb_a, wb_a, sem_ea)

            gather(srcb_b, rows_b, sem_gb).wait()

            @pl.when(nxt)
            def _():
                wait_edges(i0 + 2, srcb_a, wb_a, sem_ea)
                scatter(i0, rows_a, sem_sa).wait()
                gather(srcb_a, rows_a, sem_ga).start()

            scale(rows_b, wb_b)
            scatter(i1, rows_b, sem_sb).start()

            @pl.when(nxt)
            def _():
                start_edges(i1 + 2, srcb_b, wb_b, sem_eb)

            return carry

        lax.fori_loop(0, PAIRS, pair_body, 0)
        scatter(C - 2, rows_a, sem_sa).wait()
        scatter(C - 1, rows_b, sem_sb).wait()
        plsc.subcore_barrier()
        pltpu.sync_copy(acc.at[rsl], out_hbm.at[c, rsl])

    return spmm_kernel(h, src, dst, w, zeros)


ROW_BLK = 2000


def _linear1(x, W, b):
    def body(x_ref, w_ref, b_ref, o_ref):
        o_ref[...] = lax.dot_general(
            x_ref[...], w_ref[...], (((1,), (1,)), ((), ())),
            preferred_element_type=jnp.float32,
        ) + b_ref[...]

    return pl.pallas_call(
        body,
        grid=(N_NODES // ROW_BLK,),
        in_specs=[
            pl.BlockSpec((ROW_BLK, D), lambda i: (i, 0)),
            pl.BlockSpec((D, D), lambda i: (0, 0)),
            pl.BlockSpec((1, D), lambda i: (0, 0)),
        ],
        out_specs=pl.BlockSpec((ROW_BLK, D), lambda i: (i, 0)),
        out_shape=jax.ShapeDtypeStruct((N_NODES, D), jnp.float32),
    )(x, W, b.reshape(1, D))


def _relu_add_linear(p, W, b):
    def body(p0_ref, p1_ref, w_ref, b_ref, o_ref):
        h = jnp.maximum(p0_ref[...] + p1_ref[...], 0.0)
        o_ref[...] = lax.dot_general(
            h, w_ref[...], (((1,), (1,)), ((), ())),
            preferred_element_type=jnp.float32,
        ) + b_ref[...]

    return pl.pallas_call(
        body,
        grid=(N_NODES // ROW_BLK,),
        in_specs=[
            pl.BlockSpec((ROW_BLK, D), lambda i: (i, 0)),
            pl.BlockSpec((ROW_BLK, D), lambda i: (i, 0)),
            pl.BlockSpec((D, D), lambda i: (0, 0)),
            pl.BlockSpec((1, D), lambda i: (0, 0)),
        ],
        out_specs=pl.BlockSpec((ROW_BLK, D), lambda i: (i, 0)),
        out_shape=jax.ShapeDtypeStruct((N_NODES, D), jnp.float32),
    )(p[0], p[1], W, b.reshape(1, D))


def _add_partials(p):
    def body(p0_ref, p1_ref, o_ref):
        o_ref[...] = p0_ref[...] + p1_ref[...]

    return pl.pallas_call(
        body,
        grid=(N_NODES // ROW_BLK,),
        in_specs=[
            pl.BlockSpec((ROW_BLK, D), lambda i: (i, 0)),
            pl.BlockSpec((ROW_BLK, D), lambda i: (i, 0)),
        ],
        out_specs=pl.BlockSpec((ROW_BLK, D), lambda i: (i, 0)),
        out_shape=jax.ShapeDtypeStruct((N_NODES, D), jnp.float32),
    )(p[0], p[1])


def kernel(x, edge_index, edge_weight, W1, b1, W2, b2):
    pad = ((0, 0), (0, E_TILE_PAD - E_PER_TILE))
    src = jnp.pad(edge_index[0].astype(jnp.int32).reshape(NW, E_PER_TILE),
                  pad).reshape(NW, C, K)
    dst = jnp.pad(edge_index[1].astype(jnp.int32).reshape(NW, E_PER_TILE),
                  pad).reshape(NW, C, K)
    w = jnp.pad(edge_weight.astype(jnp.float32).reshape(NW, E_PER_TILE),
                pad).reshape(NW, C, K)
    zeros = jnp.zeros((N_PAD, D), jnp.float32)

    h = _linear1(x, W1, b1)
    p1 = _spmm_sc(jnp.stack([h] * 16), src, dst, w, zeros)
    h2 = _relu_add_linear((p1[0, :N_NODES], p1[1, :N_NODES]), W2, b2)
    p2 = _spmm_sc(jnp.stack([h2] * 16), src, dst, w, zeros)
    return _add_partials((p2[0, :N_NODES], p2[1, :N_NODES]))


# speedup vs baseline: 2.1909x; 1.0361x over previous
import functools

import jax
import jax.numpy as jnp
from jax import lax
from jax.experimental import pallas as pl
from jax.experimental.pallas import tpu as pltpu
from jax.experimental.pallas import tpu_sc as plsc

N_NODES = 10000
N_EDGES = 320000
D = 128

NC = 2
NS = 16
NW = NC * NS

E_PER_TILE = N_EDGES // NW
K = 128
C = 80
E_TILE_PAD = C * K
N_PAD = 10240
ROWS_PER_SUB = N_PAD // NS
PAIRS = C // 2


def _spmm_sc(h, src, dst, w, zeros):
    mesh = plsc.VectorSubcoreMesh(
        core_axis_name="c", subcore_axis_name="s", num_cores=NC, num_subcores=NS
    )

    @functools.partial(
        pl.kernel,
        mesh=mesh,
        out_type=jax.ShapeDtypeStruct((NC, N_PAD, D), jnp.float32),
        scratch_types=[
            pltpu.VMEM((C, K), jnp.int32),
            pltpu.VMEM((K,), jnp.int32),
            pltpu.VMEM((K,), jnp.int32),
            pltpu.VMEM((K,), jnp.float32),
            pltpu.VMEM((K,), jnp.float32),
            pltpu.VMEM((K, D), jnp.float32),
            pltpu.VMEM((K, D), jnp.float32),
            pltpu.VMEM_SHARED((N_PAD, D), jnp.float32),
            pltpu.SemaphoreType.DMA,
            pltpu.SemaphoreType.DMA,
            pltpu.SemaphoreType.DMA,
            pltpu.SemaphoreType.DMA,
            pltpu.SemaphoreType.DMA,
            pltpu.SemaphoreType.DMA,
        ],
    )
    def spmm_kernel(h_hbm, src_hbm, dst_hbm, w_hbm, z_hbm, out_hbm,
                    dst_v, srcb_a, srcb_b, wb_a, wb_b, rows_a, rows_b,
                    acc, sem_ea, sem_eb, sem_ga, sem_gb, sem_sa, sem_sb):
        c = lax.axis_index("c")
        s = lax.axis_index("s")
        tid = s * NC + c

        pltpu.sync_copy(dst_hbm.at[tid], dst_v)
        rsl = pl.ds(s * ROWS_PER_SUB, ROWS_PER_SUB)
        pltpu.sync_copy(z_hbm.at[rsl], acc.at[rsl])
        plsc.subcore_barrier()

        def edge_copies(i, srcb, wb, sem):
            return (pltpu.make_async_copy(src_hbm.at[tid, i], srcb, sem),
                    pltpu.make_async_copy(w_hbm.at[tid, i], wb, sem))

        def start_edges(i, srcb, wb, sem):
            for cp in edge_copies(i, srcb, wb, sem):
                cp.start()

        def wait_edges(i, srcb, wb, sem):
            for cp in edge_copies(i, srcb, wb, sem):
                cp.wait()

        hcopy = c * 8 + lax.rem(s, 8)

        def gather(srcb, rows, sem):
            return pltpu.make_async_copy(h_hbm.at[hcopy].at[srcb], rows, sem)

        def scale(rows, wb):
            def group_body(g, carry):
                w16 = wb[pl.ds(g * 16, 16)]
                for eo in range(16):
                    e = g * 16 + eo
                    wv = w16[eo]
                    for d0 in range(D // 16):
                        sl = pl.ds(d0 * 16, 16)
                        rows[e, sl] = rows[e, sl] * wv
                return carry

            lax.fori_loop(0, K // 16, group_body, 0)

        class _Scatter:
            def __init__(self, i, rows, sem):
                self._cp = pltpu.make_async_copy(rows, acc.at[dst_v.at[i]],
                                                 sem)

            def start(self):
                self._cp.start(add=True)

            def wait(self):
                self._cp.wait()

        scatter = _Scatter

        start_edges(0, srcb_a, wb_a, sem_ea)
        start_edges(1, srcb_b, wb_b, sem_eb)
        wait_edges(0, srcb_a, wb_a, sem_ea)
        gather(srcb_a, rows_a, sem_ga).start()

        def pair_body(j, carry):
            i0 = 2 * j
            i1 = i0 + 1
            nxt = j + 1 < PAIRS

            gather(srcb_a, rows_a, sem_ga).wait()
            wait_edges(i1, srcb_b, wb_b, sem_eb)

            @pl.when(j > 0)
            def _():
                scatter(i1 - 2, rows_b, sem_sb).wait()

            gather(srcb_b, rows_b, sem_gb).start()
            scale(rows_a, wb_a)
            scatter(i0, rows_a, sem_sa).start()

            @pl.when(nxt)
            def _():
                start_edges(i0 + 2, srcb_a, wb_a, sem_ea)

            gather(srcb_b, rows_b, sem_gb).wait()

            @pl.when(nxt)
            def _():
                wait_edges(i0 + 2, srcb_a, wb_a, sem_ea)
                scatter(i0, rows_a, sem_sa).wait()
                gather(srcb_a, rows_a, sem_ga).start()

            scale(rows_b, wb_b)
            scatter(i1, rows_b, sem_sb).start()

            @pl.when(nxt)
            def _():
                start_edges(i1 + 2, srcb_b, wb_b, sem_eb)

            return carry

        lax.fori_loop(0, PAIRS, pair_body, 0)
        scatter(C - 2, rows_a, sem_sa).wait()
        scatter(C - 1, rows_b, sem_sb).wait()
        plsc.subcore_barrier()
        pltpu.sync_copy(acc.at[rsl], out_hbm.at[c, rsl])

    return spmm_kernel(h, src, dst, w, zeros)


ROW_BLK = 2000


NCOPIES = 16


def _linear1(x, W, b):
    def body(x_ref, w_ref, b_ref, o_ref):
        y = lax.dot_general(
            x_ref[...], w_ref[...], (((1,), (1,)), ((), ())),
            preferred_element_type=jnp.float32,
        ) + b_ref[...]
        o_ref[...] = jnp.broadcast_to(y[None], (NCOPIES, ROW_BLK, D))

    return pl.pallas_call(
        body,
        grid=(N_NODES // ROW_BLK,),
        in_specs=[
            pl.BlockSpec((ROW_BLK, D), lambda i: (i, 0)),
            pl.BlockSpec((D, D), lambda i: (0, 0)),
            pl.BlockSpec((1, D), lambda i: (0, 0)),
        ],
        out_specs=pl.BlockSpec((NCOPIES, ROW_BLK, D), lambda i: (0, i, 0)),
        out_shape=jax.ShapeDtypeStruct((NCOPIES, N_NODES, D), jnp.float32),
    )(x, W, b.reshape(1, D))


def _relu_add_linear(p, W, b):
    def body(p0_ref, p1_ref, w_ref, b_ref, o_ref):
        h = jnp.maximum(p0_ref[...] + p1_ref[...], 0.0)
        y = lax.dot_general(
            h, w_ref[...], (((1,), (1,)), ((), ())),
            preferred_element_type=jnp.float32,
        ) + b_ref[...]
        o_ref[...] = jnp.broadcast_to(y[None], (NCOPIES, ROW_BLK, D))

    return pl.pallas_call(
        body,
        grid=(N_NODES // ROW_BLK,),
        in_specs=[
            pl.BlockSpec((ROW_BLK, D), lambda i: (i, 0)),
            pl.BlockSpec((ROW_BLK, D), lambda i: (i, 0)),
            pl.BlockSpec((D, D), lambda i: (0, 0)),
            pl.BlockSpec((1, D), lambda i: (0, 0)),
        ],
        out_specs=pl.BlockSpec((NCOPIES, ROW_BLK, D), lambda i: (0, i, 0)),
        out_shape=jax.ShapeDtypeStruct((NCOPIES, N_NODES, D), jnp.float32),
    )(p[0], p[1], W, b.reshape(1, D))


def _add_partials(p):
    def body(p0_ref, p1_ref, o_ref):
        o_ref[...] = p0_ref[...] + p1_ref[...]

    return pl.pallas_call(
        body,
        grid=(N_NODES // ROW_BLK,),
        in_specs=[
            pl.BlockSpec((ROW_BLK, D), lambda i: (i, 0)),
            pl.BlockSpec((ROW_BLK, D), lambda i: (i, 0)),
        ],
        out_specs=pl.BlockSpec((ROW_BLK, D), lambda i: (i, 0)),
        out_shape=jax.ShapeDtypeStruct((N_NODES, D), jnp.float32),
    )(p[0], p[1])


def kernel(x, edge_index, edge_weight, W1, b1, W2, b2):
    pad = ((0, 0), (0, E_TILE_PAD - E_PER_TILE))
    src = jnp.pad(edge_index[0].astype(jnp.int32).reshape(NW, E_PER_TILE),
                  pad).reshape(NW, C, K)
    dst = jnp.pad(edge_index[1].astype(jnp.int32).reshape(NW, E_PER_TILE),
                  pad).reshape(NW, C, K)
    w = jnp.pad(edge_weight.astype(jnp.float32).reshape(NW, E_PER_TILE),
                pad).reshape(NW, C, K)
    zeros = jnp.zeros((N_PAD, D), jnp.float32)

    h = _linear1(x, W1, b1)
    p1 = _spmm_sc(h, src, dst, w, zeros)
    h2 = _relu_add_linear((p1[0, :N_NODES], p1[1, :N_NODES]), W2, b2)
    p2 = _spmm_sc(h2, src, dst, w, zeros)
    return _add_partials((p2[0, :N_NODES], p2[1, :N_NODES]))
